# baseline (speedup 1.0000x reference)
import functools

import jax
import jax.numpy as jnp
from jax import lax
from jax.experimental import pallas as pl
from jax.experimental.pallas import tpu as pltpu
from jax.experimental.pallas import tpu_sc as plsc

P = 18
PH = 224
H, W = 256, 512
NPIX = H * W
CH = 64
NCHUNK = NPIX // CH
NC, NS = 2, 16
NW = NC * NS
CPW = NCHUNK // NW
BM_W = 40
GSLOT = 4 * CH
THR = 1e-5
EPS = 1e-12


def _sc_kernel(table, wf, x0f, y0f, bitmap, out,
               G, idxb, x0b, y0b, wb, bmall, acc, den,
               sem_aux, sem_g, sem_out):
    wid = lax.axis_index("s") * NC + lax.axis_index("c")
    lane = lax.iota(jnp.int32, 16)
    zf = jnp.zeros((16,), jnp.float32)

    pltpu.sync_copy(bitmap.at[pl.ds(wid * (CPW * BM_W), CPW * BM_W)], bmall)

    def bit_of(ci, p):
        return bmall[pl.ds(ci * BM_W + p, 16)][0]

    def fire_aux(ci):
        base = (wid + ci * NW) * CH
        qa = lax.rem(ci, 3)

        def pb_(p, _):
            @pl.when(bit_of(ci, p) != 0)
            def _():
                s = (qa * P + p) * CH
                pltpu.async_copy(x0f.at[pl.ds(p * NPIX + base, CH)],
                                 x0b.at[pl.ds(s, CH)], sem_aux)
                pltpu.async_copy(y0f.at[pl.ds(p * NPIX + base, CH)],
                                 y0b.at[pl.ds(s, CH)], sem_aux)
                pltpu.async_copy(wf.at[pl.ds((p * NPIX + base) * 4, CH * 4)],
                                 wb.at[pl.ds(s * 4, CH * 4)], sem_aux)

        lax.fori_loop(0, P, pb_, None)

    def fire_gathers(ci, q):
        qa = lax.rem(ci, 3)

        def pb_(p, _):
            @pl.when(bit_of(ci, p) != 0)
            def _():
                s = (qa * P + p) * CH
                pltpu.make_async_copy(x0f.at[pl.ds(0, CH)],
                                      x0b.at[pl.ds(s, CH)], sem_aux).wait()
                pltpu.make_async_copy(y0f.at[pl.ds(0, CH)],
                                      y0b.at[pl.ds(s, CH)], sem_aux).wait()
                pltpu.make_async_copy(wf.at[pl.ds(0, CH * 4)],
                                      wb.at[pl.ds(s * 4, CH * 4)], sem_aux).wait()
                r = (q * P + p) * 2
                for v in range(4):
                    xx0 = x0b[pl.ds(s + 16 * v, 16)]
                    yy0 = y0b[pl.ds(s + 16 * v, 16)]
                    xc0 = xx0 * P + p
                    xc1 = jnp.minimum(xx0 + 1, PH - 1) * P + p
                    yr0 = yy0 * (PH * P)
                    yr1 = jnp.minimum(yy0 + 1, PH - 1) * (PH * P)
                    idxb[r, pl.ds(16 * v, 16)] = yr0 + xc0
                    idxb[r, pl.ds(CH + 16 * v, 16)] = yr1 + xc0
                    idxb[r + 1, pl.ds(16 * v, 16)] = yr0 + xc1
                    idxb[r + 1, pl.ds(CH + 16 * v, 16)] = yr1 + xc1
                g = (q * P + p) * GSLOT
                pltpu.async_copy(table.at[idxb.at[r]],
                                 G.at[pl.ds(g, 2 * CH)], sem_g)
                pltpu.async_copy(table.at[idxb.at[r + 1]],
                                 G.at[pl.ds(g + 2 * CH, 2 * CH)], sem_g)

        lax.fori_loop(0, P, pb_, None)

    def compute(ci, q):
        chunk = wid + ci * NW
        qa = lax.rem(ci, 3)
        a0 = q * (8 * CH)
        d0 = q * CH
        for c in range(8):
            for v in range(4):
                acc[pl.ds(a0 + c * CH + 16 * v, 16)] = zf
        for v in range(4):
            den[pl.ds(d0 + 16 * v, 16)] = zf

        def pb_(p, _):
            @pl.when(bit_of(ci, p) != 0)
            def _():
                g = (q * P + p) * GSLOT
                pltpu.make_async_copy(table.at[pl.ds(0, 2 * CH)],
                                      G.at[pl.ds(g, 2 * CH)], sem_g).wait()
                pltpu.make_async_copy(table.at[pl.ds(0, 2 * CH)],
                                      G.at[pl.ds(g + 2 * CH, 2 * CH)],
                                      sem_g).wait()
                s = (qa * P + p) * CH
                for v in range(4):
                    pix = lane + (16 * v)
                    tw = []
                    for k in range(4):
                        wk = plsc.load_gather(wb, [(s + pix) * 4 + k])
                        tw.append(jnp.where(wk > THR, wk, 0.0))
                    dn = pl.ds(d0 + 16 * v, 16)
                    den[dn] = den[dn] + tw[0] + tw[1] + tw[2] + tw[3]
                    for c in range(8):
                        cv = jnp.full((16,), c, jnp.int32)
                        o = pl.ds(a0 + c * CH + 16 * v, 16)
                        sacc = acc[o]
                        for k in range(4):
                            val = plsc.load_gather(G, [g + k * CH + pix, cv])
                            sacc = sacc + tw[k] * val
                        acc[o] = sacc

        lax.fori_loop(0, P, pb_, None)

        for v in range(4):
            dv = jnp.maximum(den[pl.ds(d0 + 16 * v, 16)], EPS)
            for c in range(8):
                o = pl.ds(a0 + c * CH + 16 * v, 16)
                acc[o] = acc[o] / dv
        pltpu.async_copy(acc.at[pl.ds(a0, 8 * CH)],
                         out.at[pl.ds(chunk * (8 * CH), 8 * CH)], sem_out)

    fire_aux(0)
    fire_gathers(0, 0)

    def main_body(i, _):
        @pl.when(i < CPW)
        def _():
            @pl.when(i > 0)
            def _():
                fire_gathers(i, lax.rem(i, 2))

            @pl.when(i + 1 < CPW)
            def _():
                fire_aux(i + 1)

        @pl.when(i > 0)
        def _():
            @pl.when(i > 2)
            def _():
                pltpu.make_async_copy(acc.at[pl.ds(0, 8 * CH)],
                                      out.at[pl.ds(0, 8 * CH)], sem_out).wait()

            compute(i - 1, lax.rem(i - 1, 2))

    lax.fori_loop(0, CPW + 1, main_body, None)
    for _ in range(2):
        pltpu.make_async_copy(acc.at[pl.ds(0, 8 * CH)],
                              out.at[pl.ds(0, 8 * CH)], sem_out).wait()


_pers2equi_sc = functools.partial(
    pl.kernel,
    out_type=jax.ShapeDtypeStruct((NCHUNK * 8 * CH,), jnp.float32),
    mesh=plsc.VectorSubcoreMesh(core_axis_name="c", subcore_axis_name="s"),
    compiler_params=pltpu.CompilerParams(needs_layout_passes=False,
                                         use_tc_tiling_on_sc=False),
    scratch_types=[
        pltpu.VMEM((2 * P * GSLOT, 8), jnp.float32),
        pltpu.VMEM((2 * P * 2, 2 * CH), jnp.int32),
        pltpu.VMEM((3 * P * CH,), jnp.int32),
        pltpu.VMEM((3 * P * CH,), jnp.int32),
        pltpu.VMEM((3 * P * CH * 4,), jnp.float32),
        pltpu.VMEM((CPW * BM_W,), jnp.int32),
        pltpu.VMEM((2 * 8 * CH,), jnp.float32),
        pltpu.VMEM((2 * CH,), jnp.float32),
        pltpu.SemaphoreType.DMA,
        pltpu.SemaphoreType.DMA,
        pltpu.SemaphoreType.DMA,
    ],
)(_sc_kernel)


def kernel(x, w_list, mask, x0, y0, x1, y1):
    del x1, y1
    table = x[0].reshape(8, PH * PH * P).T
    wf = w_list.reshape(P * NPIX * 4)
    x0f = x0.astype(jnp.int32).reshape(P * NPIX)
    y0f = y0.astype(jnp.int32).reshape(P * NPIX)
    bm = mask.reshape(P, NCHUNK, CH).max(-1).astype(jnp.int32)
    bm = jnp.pad(bm.T, ((0, 0), (0, BM_W - P)))
    bm = bm.reshape(CPW, NW, BM_W).transpose(1, 0, 2).reshape(NW * CPW * BM_W)
    outf = _pers2equi_sc(table, wf, x0f, y0f, bm)
    out = outf.reshape(NCHUNK, 8, CH).transpose(1, 0, 2)
    return out.reshape(1, 8, H, W)


# speedup vs baseline: 1.0497x; 1.0497x over previous
import functools

import jax
import jax.numpy as jnp
from jax import lax
from jax.experimental import pallas as pl
from jax.experimental.pallas import tpu as pltpu
from jax.experimental.pallas import tpu_sc as plsc

P = 18
PH = 224
H, W = 256, 512
NPIX = H * W
NT = PH * PH * P
CH = 64
NCHUNK = NPIX // CH
NC, NS = 2, 16
NW = NC * NS
CPW = NCHUNK // NW
GSLOT = 4 * CH
THR = 1e-5
EPS = 1e-12

TBN = 7168


def _tp_body(x_ref, o_ref):
    o_ref[...] = x_ref[...].T


_transpose_tc = pl.pallas_call(
    _tp_body,
    grid=(NT // TBN,),
    in_specs=[pl.BlockSpec((8, TBN), lambda i: (0, i))],
    out_specs=pl.BlockSpec((TBN, 8), lambda i: (i, 0)),
    out_shape=jax.ShapeDtypeStruct((NT, 8), jnp.float32),
)


def _sc_kernel(table, wf, x0f, y0f, bmpack, out,
               G, idxb, x0b, y0b, wb, bmw, acc, den,
               sem_aux, sem_g, sem_out):
    wid = lax.axis_index("s") * NC + lax.axis_index("c")
    lane = lax.iota(jnp.int32, 16)
    zf = jnp.zeros((16,), jnp.float32)

    pltpu.sync_copy(bmpack.at[pl.ds(0, NCHUNK)], bmw.at[pl.ds(0, NCHUNK)])

    def word_of(ci):
        return bmw[pl.ds(wid + ci * NW, 16)][0]

    def fire_aux(ci):
        base = (wid + ci * NW) * CH
        qa = lax.rem(ci, 3)
        word = word_of(ci)

        def pb_(p, _):
            @pl.when((word >> p) & 1 != 0)
            def _():
                s = (qa * P + p) * CH
                pltpu.async_copy(x0f.at[pl.ds(p * NPIX + base, CH)],
                                 x0b.at[pl.ds(s, CH)], sem_aux)
                pltpu.async_copy(y0f.at[pl.ds(p * NPIX + base, CH)],
                                 y0b.at[pl.ds(s, CH)], sem_aux)
                pltpu.async_copy(wf.at[pl.ds((p * NPIX + base) * 4, CH * 4)],
                                 wb.at[pl.ds(s * 4, CH * 4)], sem_aux)

        lax.fori_loop(0, P, pb_, None)

    def fire_gathers(ci, q):
        qa = lax.rem(ci, 3)
        word = word_of(ci)

        def pb_(p, _):
            @pl.when((word >> p) & 1 != 0)
            def _():
                s = (qa * P + p) * CH
                pltpu.make_async_copy(x0f.at[pl.ds(0, CH)],
                                      x0b.at[pl.ds(s, CH)], sem_aux).wait()
                pltpu.make_async_copy(y0f.at[pl.ds(0, CH)],
                                      y0b.at[pl.ds(s, CH)], sem_aux).wait()
                pltpu.make_async_copy(wf.at[pl.ds(0, CH * 4)],
                                      wb.at[pl.ds(s * 4, CH * 4)], sem_aux).wait()
                r = (q * P + p) * 2
                for v in range(4):
                    xx0 = x0b[pl.ds(s + 16 * v, 16)]
                    yy0 = y0b[pl.ds(s + 16 * v, 16)]
                    xc0 = xx0 * P + p
                    xc1 = jnp.minimum(xx0 + 1, PH - 1) * P + p
                    yr0 = yy0 * (PH * P)
                    yr1 = jnp.minimum(yy0 + 1, PH - 1) * (PH * P)
                    idxb[r, pl.ds(16 * v, 16)] = yr0 + xc0
                    idxb[r, pl.ds(CH + 16 * v, 16)] = yr1 + xc0
                    idxb[r + 1, pl.ds(16 * v, 16)] = yr0 + xc1
                    idxb[r + 1, pl.ds(CH + 16 * v, 16)] = yr1 + xc1
                g = (q * P + p) * GSLOT
                pltpu.async_copy(table.at[idxb.at[r]],
                                 G.at[pl.ds(g, 2 * CH)], sem_g)
                pltpu.async_copy(table.at[idxb.at[r + 1]],
                                 G.at[pl.ds(g + 2 * CH, 2 * CH)], sem_g)

        lax.fori_loop(0, P, pb_, None)

    def compute(ci, q):
        base = (wid + ci * NW) * CH
        qa = lax.rem(ci, 3)
        word = word_of(ci)
        a0 = q * (8 * CH)
        d0 = q * CH
        for c in range(8):
            for v in range(4):
                acc[pl.ds(a0 + c * CH + 16 * v, 16)] = zf
        for v in range(4):
            den[pl.ds(d0 + 16 * v, 16)] = zf

        def pb_(p, _):
            @pl.when((word >> p) & 1 != 0)
            def _():
                g = (q * P + p) * GSLOT
                pltpu.make_async_copy(table.at[pl.ds(0, 2 * CH)],
                                      G.at[pl.ds(g, 2 * CH)], sem_g).wait()
                pltpu.make_async_copy(table.at[pl.ds(0, 2 * CH)],
                                      G.at[pl.ds(g + 2 * CH, 2 * CH)],
                                      sem_g).wait()
                s = (qa * P + p) * CH
                for v in range(4):
                    pix = lane + (16 * v)
                    tw = []
                    for k in range(4):
                        wk = plsc.load_gather(wb, [(s + pix) * 4 + k])
                        tw.append(jnp.where(wk > THR, wk, 0.0))
                    dn = pl.ds(d0 + 16 * v, 16)
                    den[dn] = den[dn] + tw[0] + tw[1] + tw[2] + tw[3]
                    for c in range(8):
                        cv = jnp.full((16,), c, jnp.int32)
                        o = pl.ds(a0 + c * CH + 16 * v, 16)
                        sacc = acc[o]
                        for k in range(4):
                            val = plsc.load_gather(G, [g + k * CH + pix, cv])
                            sacc = sacc + tw[k] * val
                        acc[o] = sacc

        lax.fori_loop(0, P, pb_, None)

        for v in range(4):
            dv = jnp.maximum(den[pl.ds(d0 + 16 * v, 16)], EPS)
            for c in range(8):
                o = pl.ds(a0 + c * CH + 16 * v, 16)
                acc[o] = acc[o] / dv
        for c in range(8):
            pltpu.async_copy(acc.at[pl.ds(a0 + c * CH, CH)],
                             out.at[pl.ds(c * NPIX + base, CH)], sem_out)

    def drain_out():
        for _ in range(8):
            pltpu.make_async_copy(acc.at[pl.ds(0, CH)],
                                  out.at[pl.ds(0, CH)], sem_out).wait()

    fire_aux(0)
    fire_gathers(0, 0)

    def main_body(i, _):
        @pl.when(i < CPW)
        def _():
            @pl.when(i > 0)
            def _():
                fire_gathers(i, lax.rem(i, 2))

            @pl.when(i + 1 < CPW)
            def _():
                fire_aux(i + 1)

        @pl.when(i > 0)
        def _():
            @pl.when(i > 2)
            def _():
                drain_out()

            compute(i - 1, lax.rem(i - 1, 2))

    lax.fori_loop(0, CPW + 1, main_body, None)
    drain_out()
    drain_out()


_pers2equi_sc = functools.partial(
    pl.kernel,
    out_type=jax.ShapeDtypeStruct((8 * NPIX,), jnp.float32),
    mesh=plsc.VectorSubcoreMesh(core_axis_name="c", subcore_axis_name="s"),
    compiler_params=pltpu.CompilerParams(needs_layout_passes=False,
                                         use_tc_tiling_on_sc=False),
    scratch_types=[
        pltpu.VMEM((2 * P * GSLOT, 8), jnp.float32),
        pltpu.VMEM((2 * P * 2, 2 * CH), jnp.int32),
        pltpu.VMEM((3 * P * CH,), jnp.int32),
        pltpu.VMEM((3 * P * CH,), jnp.int32),
        pltpu.VMEM((3 * P * CH * 4,), jnp.float32),
        pltpu.VMEM((NCHUNK + 16,), jnp.int32),
        pltpu.VMEM((2 * 8 * CH,), jnp.float32),
        pltpu.VMEM((2 * CH,), jnp.float32),
        pltpu.SemaphoreType.DMA,
        pltpu.SemaphoreType.DMA,
        pltpu.SemaphoreType.DMA,
    ],
)(_sc_kernel)


def kernel(x, w_list, mask, x0, y0, x1, y1):
    del x1, y1
    table = _transpose_tc(x[0].reshape(8, NT))
    wf = w_list.reshape(P * NPIX * 4)
    x0f = x0.astype(jnp.int32).reshape(P * NPIX)
    y0f = y0.astype(jnp.int32).reshape(P * NPIX)
    bits = mask.astype(jnp.int32).reshape(P, NCHUNK, CH).max(-1)
    bmpack = (bits << jnp.arange(P, dtype=jnp.int32)[:, None]).sum(0)
    outf = _pers2equi_sc(table, wf, x0f, y0f, bmpack)
    return outf.reshape(1, 8, H, W)


# speedup vs baseline: 3.0139x; 2.8713x over previous
import functools

import jax
import jax.numpy as jnp
from jax import lax
from jax.experimental import pallas as pl
from jax.experimental.pallas import tpu as pltpu
from jax.experimental.pallas import tpu_sc as plsc

P = 18
PH = 224
H, W = 256, 512
NPIX = H * W
NT = PH * PH * P
CH = 64
NCHUNK = NPIX // CH
NC, NS = 2, 16
NW = NC * NS
CPW = NCHUNK // NW
GSLOT = 4 * CH
THR = 1e-5
EPS = 1e-12

TBN = 7168


def _tp_body(x_ref, o_ref):
    o_ref[...] = x_ref[...].T


_transpose_tc = pl.pallas_call(
    _tp_body,
    grid=(NT // TBN,),
    in_specs=[pl.BlockSpec((8, TBN), lambda i: (0, i))],
    out_specs=pl.BlockSpec((TBN, 8), lambda i: (i, 0)),
    out_shape=jax.ShapeDtypeStruct((NT, 8), jnp.float32),
)


def _sc_kernel(table, wf, x0f, y0f, bmpack, out,
               G, idxb, x0b, y0b, wb, bmw, acc, den,
               sem_aux, sem_g, sem_out):
    wid = lax.axis_index("s") * NC + lax.axis_index("c")
    lane = lax.iota(jnp.int32, 16)
    zf = jnp.zeros((16,), jnp.float32)

    pltpu.sync_copy(bmpack.at[pl.ds(0, NCHUNK)], bmw.at[pl.ds(0, NCHUNK)])

    def word_of(ci):
        return bmw[pl.ds(wid + ci * NW, 16)][0]

    def fire_aux(ci):
        chunk = wid + ci * NW
        base = chunk * CH
        row = chunk >> 3
        s8 = chunk & 7
        jhi = s8 >> 1
        jlo = (s8 & 1) * CH
        qa = lax.rem(ci, 3)
        word = word_of(ci)

        def pb_(p, _):
            @pl.when((word >> p) & 1 != 0)
            def _():
                s = (qa * P + p) * CH
                pltpu.async_copy(x0f.at[pl.ds(p * NPIX + base, CH)],
                                 x0b.at[pl.ds(s, CH)], sem_aux)
                pltpu.async_copy(y0f.at[pl.ds(p * NPIX + base, CH)],
                                 y0b.at[pl.ds(s, CH)], sem_aux)
                woff = ((p * H + row) * 4 + jhi) * 512 + jlo
                for k in range(4):
                    pltpu.async_copy(wf.at[pl.ds(woff + k * 128, CH)],
                                     wb.at[pl.ds(s * 4 + k * CH, CH)], sem_aux)

        lax.fori_loop(0, P, pb_, None)

    def fire_gathers(ci, q):
        qa = lax.rem(ci, 3)
        word = word_of(ci)

        def pb_(p, _):
            @pl.when((word >> p) & 1 != 0)
            def _():
                s = (qa * P + p) * CH
                pltpu.make_async_copy(x0f.at[pl.ds(0, CH)],
                                      x0b.at[pl.ds(s, CH)], sem_aux).wait()
                pltpu.make_async_copy(y0f.at[pl.ds(0, CH)],
                                      y0b.at[pl.ds(s, CH)], sem_aux).wait()
                for k in range(4):
                    pltpu.make_async_copy(wf.at[pl.ds(0, CH)],
                                          wb.at[pl.ds(s * 4 + k * CH, CH)],
                                          sem_aux).wait()
                r = (q * P + p) * 2
                pb = p * (PH * PH)
                for v in range(4):
                    xx0 = x0b[pl.ds(s + 16 * v, 16)]
                    yy0 = y0b[pl.ds(s + 16 * v, 16)]
                    xc0 = pb + xx0
                    xc1 = pb + jnp.minimum(xx0 + 1, PH - 1)
                    yr0 = yy0 * PH
                    yr1 = jnp.minimum(yy0 + 1, PH - 1) * PH
                    idxb[r, pl.ds(16 * v, 16)] = yr0 + xc0
                    idxb[r, pl.ds(CH + 16 * v, 16)] = yr1 + xc0
                    idxb[r + 1, pl.ds(16 * v, 16)] = yr0 + xc1
                    idxb[r + 1, pl.ds(CH + 16 * v, 16)] = yr1 + xc1
                g = (q * P + p) * GSLOT
                pltpu.async_copy(table.at[idxb.at[r]],
                                 G.at[pl.ds(g, 2 * CH)], sem_g)
                pltpu.async_copy(table.at[idxb.at[r + 1]],
                                 G.at[pl.ds(g + 2 * CH, 2 * CH)], sem_g)

        lax.fori_loop(0, P, pb_, None)

    def compute(ci, q):
        chunk = wid + ci * NW
        row = chunk >> 3
        s8 = chunk & 7
        obase = ((row >> 3) * 4 + (s8 >> 1)) * 1024 + (row & 7) * 128 + (s8 & 1) * CH
        qa = lax.rem(ci, 3)
        word = word_of(ci)
        a0 = q * (8 * CH)
        d0 = q * CH
        for c in range(8):
            for v in range(4):
                acc[pl.ds(a0 + c * CH + 16 * v, 16)] = zf
        for v in range(4):
            den[pl.ds(d0 + 16 * v, 16)] = zf

        def pb_(p, _):
            @pl.when((word >> p) & 1 != 0)
            def _():
                g = (q * P + p) * GSLOT
                pltpu.make_async_copy(table.at[pl.ds(0, 2 * CH)],
                                      G.at[pl.ds(g, 2 * CH)], sem_g).wait()
                pltpu.make_async_copy(table.at[pl.ds(0, 2 * CH)],
                                      G.at[pl.ds(g + 2 * CH, 2 * CH)],
                                      sem_g).wait()
                s = (qa * P + p) * CH
                for v in range(4):
                    pix = lane + (16 * v)
                    tw = []
                    for k in range(4):
                        wk = wb[pl.ds(s * 4 + k * CH + 16 * v, 16)]
                        tw.append(jnp.where(wk > THR, wk, 0.0))
                    dn = pl.ds(d0 + 16 * v, 16)
                    den[dn] = den[dn] + tw[0] + tw[1] + tw[2] + tw[3]
                    for c in range(8):
                        cv = jnp.full((16,), c, jnp.int32)
                        o = pl.ds(a0 + c * CH + 16 * v, 16)
                        sacc = acc[o]
                        for k in range(4):
                            val = plsc.load_gather(G, [g + k * CH + pix, cv])
                            sacc = sacc + tw[k] * val
                        acc[o] = sacc

        lax.fori_loop(0, P, pb_, None)

        for v in range(4):
            dv = jnp.maximum(den[pl.ds(d0 + 16 * v, 16)], EPS)
            for c in range(8):
                o = pl.ds(a0 + c * CH + 16 * v, 16)
                acc[o] = acc[o] / dv
        for c in range(8):
            pltpu.async_copy(acc.at[pl.ds(a0 + c * CH, CH)],
                             out.at[pl.ds(c * NPIX + obase, CH)], sem_out)

    def drain_out():
        for _ in range(8):
            pltpu.make_async_copy(acc.at[pl.ds(0, CH)],
                                  out.at[pl.ds(0, CH)], sem_out).wait()

    fire_aux(0)
    fire_gathers(0, 0)

    def main_body(i, _):
        @pl.when(i < CPW)
        def _():
            @pl.when(i > 0)
            def _():
                fire_gathers(i, lax.rem(i, 2))

            @pl.when(i + 1 < CPW)
            def _():
                fire_aux(i + 1)

        @pl.when(i > 0)
        def _():
            @pl.when(i > 2)
            def _():
                drain_out()

            compute(i - 1, lax.rem(i - 1, 2))

    lax.fori_loop(0, CPW + 1, main_body, None)
    drain_out()
    drain_out()


_pers2equi_sc = functools.partial(
    pl.kernel,
    out_type=jax.ShapeDtypeStruct((8 * NPIX,), jnp.float32),
    mesh=plsc.VectorSubcoreMesh(core_axis_name="c", subcore_axis_name="s"),
    compiler_params=pltpu.CompilerParams(needs_layout_passes=False,
                                         use_tc_tiling_on_sc=False),
    scratch_types=[
        pltpu.VMEM((2 * P * GSLOT, 8), jnp.float32),
        pltpu.VMEM((2 * P * 2, 2 * CH), jnp.int32),
        pltpu.VMEM((3 * P * CH,), jnp.int32),
        pltpu.VMEM((3 * P * CH,), jnp.int32),
        pltpu.VMEM((3 * P * CH * 4,), jnp.float32),
        pltpu.VMEM((NCHUNK + 16,), jnp.int32),
        pltpu.VMEM((2 * 8 * CH,), jnp.float32),
        pltpu.VMEM((2 * CH,), jnp.float32),
        pltpu.SemaphoreType.DMA,
        pltpu.SemaphoreType.DMA,
        pltpu.SemaphoreType.DMA,
    ],
)(_sc_kernel)


def kernel(x, w_list, mask, x0, y0, x1, y1):
    del x1, y1
    xv = jnp.transpose(x[0], (0, 3, 1, 2)).reshape(8, NT)
    table = _transpose_tc(xv)
    wf = jnp.transpose(w_list.reshape(P, H, 4, 128, 4),
                       (0, 1, 2, 4, 3)).reshape(P * NPIX * 4)
    x0f = x0.astype(jnp.int32).reshape(P * NPIX)
    y0f = y0.astype(jnp.int32).reshape(P * NPIX)
    bits = mask.astype(jnp.int32).reshape(P, NCHUNK, CH).max(-1)
    bmpack = (bits << jnp.arange(P, dtype=jnp.int32)[:, None]).sum(0)
    outf = _pers2equi_sc(table, wf, x0f, y0f, bmpack)
    out = outf.reshape(8, H // 8, 4, 8, 128).transpose(0, 1, 3, 2, 4)
    return out.reshape(1, 8, H, W)


# speedup vs baseline: 3.0178x; 1.0013x over previous
import functools

import jax
import jax.numpy as jnp
from jax import lax
from jax.experimental import pallas as pl
from jax.experimental.pallas import tpu as pltpu
from jax.experimental.pallas import tpu_sc as plsc

P = 18
PH = 224
H, W = 256, 512
NPIX = H * W
NT = PH * PH * P
CH = 64
NCHUNK = NPIX // CH
NC, NS = 2, 16
NW = NC * NS
CPW = NCHUNK // NW
GSLOT = 4 * CH
THR = 1e-5
EPS = 1e-12

TBN = 7168


def _tp_body(x_ref, o_ref):
    o_ref[...] = x_ref[...].T


_transpose_tc = pl.pallas_call(
    _tp_body,
    grid=(NT // TBN,),
    in_specs=[pl.BlockSpec((8, TBN), lambda i: (0, i))],
    out_specs=pl.BlockSpec((TBN, 8), lambda i: (i, 0)),
    out_shape=jax.ShapeDtypeStruct((NT, 8), jnp.float32),
)


def _sc_kernel(table, wf, x0f, y0f, bmpack, out,
               G, idxb, x0b, y0b, wb, bmw, acc, den,
               sem_aux, sem_g, sem_out):
    wid = lax.axis_index("s") * NC + lax.axis_index("c")
    lane = lax.iota(jnp.int32, 16)
    zf = jnp.zeros((16,), jnp.float32)

    pltpu.sync_copy(bmpack.at[pl.ds(0, NCHUNK)], bmw.at[pl.ds(0, NCHUNK)])

    def word_of(ci):
        return bmw[pl.ds(wid + ci * NW, 16)][0]

    def fire_aux(ci):
        chunk = wid + ci * NW
        base = chunk * CH
        row = chunk >> 3
        s8 = chunk & 7
        jhi = s8 >> 1
        jlo = (s8 & 1) * CH
        qa = lax.rem(ci, 3)
        word = word_of(ci)

        def pb_(p, _):
            @pl.when((word >> p) & 1 != 0)
            def _():
                s = (qa * P + p) * CH
                pltpu.async_copy(x0f.at[pl.ds(p * NPIX + base, CH)],
                                 x0b.at[pl.ds(s, CH)], sem_aux)
                pltpu.async_copy(y0f.at[pl.ds(p * NPIX + base, CH)],
                                 y0b.at[pl.ds(s, CH)], sem_aux)
                woff = ((p * H + row) * 4 + jhi) * 512 + jlo
                for k in range(4):
                    pltpu.async_copy(wf.at[pl.ds(woff + k * 128, CH)],
                                     wb.at[pl.ds(s * 4 + k * CH, CH)], sem_aux)

        lax.fori_loop(0, P, pb_, None)

    def fire_gathers(ci, q):
        qa = lax.rem(ci, 3)
        word = word_of(ci)

        def pb_(p, _):
            @pl.when((word >> p) & 1 != 0)
            def _():
                s = (qa * P + p) * CH
                pltpu.make_async_copy(x0f.at[pl.ds(0, CH)],
                                      x0b.at[pl.ds(s, CH)], sem_aux).wait()
                pltpu.make_async_copy(y0f.at[pl.ds(0, CH)],
                                      y0b.at[pl.ds(s, CH)], sem_aux).wait()
                for k in range(4):
                    pltpu.make_async_copy(wf.at[pl.ds(0, CH)],
                                          wb.at[pl.ds(s * 4 + k * CH, CH)],
                                          sem_aux).wait()
                r = (q * P + p) * 2
                pb = p * (PH * PH)
                for v in range(4):
                    xx0 = x0b[pl.ds(s + 16 * v, 16)]
                    yy0 = y0b[pl.ds(s + 16 * v, 16)]
                    xc0 = pb + xx0
                    xc1 = pb + jnp.minimum(xx0 + 1, PH - 1)
                    yr0 = yy0 * PH
                    yr1 = jnp.minimum(yy0 + 1, PH - 1) * PH
                    idxb[r, pl.ds(16 * v, 16)] = yr0 + xc0
                    idxb[r, pl.ds(CH + 16 * v, 16)] = yr1 + xc0
                    idxb[r + 1, pl.ds(16 * v, 16)] = yr0 + xc1
                    idxb[r + 1, pl.ds(CH + 16 * v, 16)] = yr1 + xc1
                g = (q * P + p) * GSLOT
                pltpu.async_copy(table.at[idxb.at[r]],
                                 G.at[pl.ds(g, 2 * CH)], sem_g)
                pltpu.async_copy(table.at[idxb.at[r + 1]],
                                 G.at[pl.ds(g + 2 * CH, 2 * CH)], sem_g)

        lax.fori_loop(0, P, pb_, None)

    def compute(ci, q):
        chunk = wid + ci * NW
        row = chunk >> 3
        s8 = chunk & 7
        obase = ((row >> 3) * 4 + (s8 >> 1)) * 1024 + (row & 7) * 128 + (s8 & 1) * CH
        qa = lax.rem(ci, 3)
        word = word_of(ci)
        a0 = q * (8 * CH)
        d0 = q * CH
        for c in range(8):
            for v in range(4):
                acc[pl.ds(a0 + c * CH + 16 * v, 16)] = zf
        for v in range(4):
            den[pl.ds(d0 + 16 * v, 16)] = zf

        def pb_(p, _):
            @pl.when((word >> p) & 1 != 0)
            def _():
                g = (q * P + p) * GSLOT
                pltpu.make_async_copy(table.at[pl.ds(0, 2 * CH)],
                                      G.at[pl.ds(g, 2 * CH)], sem_g).wait()
                pltpu.make_async_copy(table.at[pl.ds(0, 2 * CH)],
                                      G.at[pl.ds(g + 2 * CH, 2 * CH)],
                                      sem_g).wait()
                s = (qa * P + p) * CH
                for v in range(4):
                    pix = lane + (16 * v)
                    tw = []
                    for k in range(4):
                        wk = wb[pl.ds(s * 4 + k * CH + 16 * v, 16)]
                        tw.append(jnp.where(wk > THR, wk, 0.0))
                    dn = pl.ds(d0 + 16 * v, 16)
                    den[dn] = den[dn] + tw[0] + tw[1] + tw[2] + tw[3]
                    for c in range(8):
                        cv = jnp.full((16,), c, jnp.int32)
                        o = pl.ds(a0 + c * CH + 16 * v, 16)
                        sacc = acc[o]
                        for k in range(4):
                            val = plsc.load_gather(G, [g + k * CH + pix, cv])
                            sacc = sacc + tw[k] * val
                        acc[o] = sacc

        lax.fori_loop(0, P, pb_, None)

        for v in range(4):
            dv = jnp.maximum(den[pl.ds(d0 + 16 * v, 16)], EPS)
            for c in range(8):
                o = pl.ds(a0 + c * CH + 16 * v, 16)
                acc[o] = acc[o] / dv
        for c in range(8):
            pltpu.async_copy(acc.at[pl.ds(a0 + c * CH, CH)],
                             out.at[pl.ds(c * NPIX + obase, CH)], sem_out)

    def drain_out():
        for _ in range(8):
            pltpu.make_async_copy(acc.at[pl.ds(0, CH)],
                                  out.at[pl.ds(0, CH)], sem_out).wait()

    fire_aux(0)
    fire_gathers(0, 0)

    def main_body(i, _):
        @pl.when(i < CPW)
        def _():
            @pl.when(i > 0)
            def _():
                fire_gathers(i, lax.rem(i, 2))

            @pl.when(i + 1 < CPW)
            def _():
                fire_aux(i + 1)

        @pl.when(i > 0)
        def _():
            @pl.when(i > 2)
            def _():
                drain_out()

            compute(i - 1, lax.rem(i - 1, 2))

    lax.fori_loop(0, CPW + 1, main_body, None)
    drain_out()
    drain_out()


_pers2equi_sc = functools.partial(
    pl.kernel,
    out_type=jax.ShapeDtypeStruct((8 * NPIX,), jnp.float32),
    mesh=plsc.VectorSubcoreMesh(core_axis_name="c", subcore_axis_name="s"),
    compiler_params=pltpu.CompilerParams(needs_layout_passes=False,
                                         use_tc_tiling_on_sc=False),
    scratch_types=[
        pltpu.VMEM((2 * P * GSLOT, 8), jnp.float32),
        pltpu.VMEM((2 * P * 2, 2 * CH), jnp.int32),
        pltpu.VMEM((3 * P * CH,), jnp.int32),
        pltpu.VMEM((3 * P * CH,), jnp.int32),
        pltpu.VMEM((3 * P * CH * 4,), jnp.float32),
        pltpu.VMEM((NCHUNK + 16,), jnp.int32),
        pltpu.VMEM((2 * 8 * CH,), jnp.float32),
        pltpu.VMEM((2 * CH,), jnp.float32),
        pltpu.SemaphoreType.DMA,
        pltpu.SemaphoreType.DMA,
        pltpu.SemaphoreType.DMA,
    ],
)(_sc_kernel)


def kernel(x, w_list, mask, x0, y0, x1, y1):
    del x1, y1
    xv = jnp.transpose(x, (0, 1, 4, 2, 3))[0].reshape(8, NT)
    table = _transpose_tc(xv)
    wf = jnp.transpose(w_list.reshape(P, H, 4, 128, 4),
                       (0, 1, 2, 4, 3)).reshape(P * NPIX * 4)
    x0f = x0.astype(jnp.int32).reshape(P * NPIX)
    y0f = y0.astype(jnp.int32).reshape(P * NPIX)
    bits = mask.astype(jnp.int32).reshape(P, NCHUNK, CH).max(-1)
    bmpack = (bits << jnp.arange(P, dtype=jnp.int32)[:, None]).sum(0)
    outf = _pers2equi_sc(table, wf, x0f, y0f, bmpack)
    out = outf.reshape(8, H // 8, 4, 8, 128).transpose(0, 1, 3, 2, 4)
    return out.reshape(1, 8, H, W)


# speedup vs baseline: 4.2757x; 1.4168x over previous
import functools

import jax
import jax.numpy as jnp
from jax import lax
from jax.experimental import pallas as pl
from jax.experimental.pallas import tpu as pltpu
from jax.experimental.pallas import tpu_sc as plsc

P = 18
PH = 224
H, W = 256, 512
NPIX = H * W
NT = PH * PH * P
CH = 64
NCHUNK = NPIX // CH
NC, NS = 2, 16
NW = NC * NS
CPW = NCHUNK // NW
GSLOT = 4 * CH
THR = 1e-5
EPS = 1e-12

TYB = 8


def _tp_body(x_ref, o_ref):
    o_ref[...] = jnp.transpose(x_ref[...], (1, 2, 3, 0)).reshape(TYB * PH, 8)


_transpose_tc = pl.pallas_call(
    _tp_body,
    grid=(P, PH // TYB),
    in_specs=[pl.BlockSpec((8, 1, TYB, PH), lambda p, i: (0, p, i, 0))],
    out_specs=pl.BlockSpec((TYB * PH, 8), lambda p, i: (p * (PH // TYB) + i, 0)),
    out_shape=jax.ShapeDtypeStruct((NT, 8), jnp.float32),
)


def _sc_kernel(table, wf, x0f, y0f, bmpack, out,
               G, idxb, x0b, y0b, wb, bmw, acc, den,
               sem_aux, sem_g, sem_out):
    wid = lax.axis_index("s") * NC + lax.axis_index("c")
    lane = lax.iota(jnp.int32, 16)
    zf = jnp.zeros((16,), jnp.float32)

    pltpu.sync_copy(bmpack.at[pl.ds(0, NCHUNK)], bmw.at[pl.ds(0, NCHUNK)])

    def word_of(ci):
        return bmw[pl.ds(wid + ci * NW, 16)][0]

    def fire_aux(ci):
        chunk = wid + ci * NW
        base = chunk * CH
        row = chunk >> 3
        s8 = chunk & 7
        jhi = s8 >> 1
        jlo = (s8 & 1) * CH
        qa = lax.rem(ci, 3)
        word = word_of(ci)

        def pb_(p, _):
            @pl.when((word >> p) & 1 != 0)
            def _():
                s = (qa * P + p) * CH
                pltpu.async_copy(x0f.at[pl.ds(p * NPIX + base, CH)],
                                 x0b.at[pl.ds(s, CH)], sem_aux)
                pltpu.async_copy(y0f.at[pl.ds(p * NPIX + base, CH)],
                                 y0b.at[pl.ds(s, CH)], sem_aux)
                woff = ((p * H + row) * 4 + jhi) * 512 + jlo
                for k in range(4):
                    pltpu.async_copy(wf.at[pl.ds(woff + k * 128, CH)],
                                     wb.at[pl.ds(s * 4 + k * CH, CH)], sem_aux)

        lax.fori_loop(0, P, pb_, None)

    def fire_gathers(ci, q):
        qa = lax.rem(ci, 3)
        word = word_of(ci)

        def pb_(p, _):
            @pl.when((word >> p) & 1 != 0)
            def _():
                s = (qa * P + p) * CH
                pltpu.make_async_copy(x0f.at[pl.ds(0, CH)],
                                      x0b.at[pl.ds(s, CH)], sem_aux).wait()
                pltpu.make_async_copy(y0f.at[pl.ds(0, CH)],
                                      y0b.at[pl.ds(s, CH)], sem_aux).wait()
                for k in range(4):
                    pltpu.make_async_copy(wf.at[pl.ds(0, CH)],
                                          wb.at[pl.ds(s * 4 + k * CH, CH)],
                                          sem_aux).wait()
                r = (q * P + p) * 2
                pb = p * (PH * PH)
                for v in range(4):
                    xx0 = x0b[pl.ds(s + 16 * v, 16)]
                    yy0 = y0b[pl.ds(s + 16 * v, 16)]
                    xc0 = pb + xx0
                    xc1 = pb + jnp.minimum(xx0 + 1, PH - 1)
                    yr0 = yy0 * PH
                    yr1 = jnp.minimum(yy0 + 1, PH - 1) * PH
                    idxb[r, pl.ds(16 * v, 16)] = yr0 + xc0
                    idxb[r, pl.ds(CH + 16 * v, 16)] = yr1 + xc0
                    idxb[r + 1, pl.ds(16 * v, 16)] = yr0 + xc1
                    idxb[r + 1, pl.ds(CH + 16 * v, 16)] = yr1 + xc1
                g = (q * P + p) * GSLOT
                pltpu.async_copy(table.at[idxb.at[r]],
                                 G.at[pl.ds(g, 2 * CH)], sem_g)
                pltpu.async_copy(table.at[idxb.at[r + 1]],
                                 G.at[pl.ds(g + 2 * CH, 2 * CH)], sem_g)

        lax.fori_loop(0, P, pb_, None)

    def compute(ci, q):
        chunk = wid + ci * NW
        row = chunk >> 3
        s8 = chunk & 7
        obase = ((row >> 3) * 4 + (s8 >> 1)) * 1024 + (row & 7) * 128 + (s8 & 1) * CH
        qa = lax.rem(ci, 3)
        word = word_of(ci)
        a0 = q * (8 * CH)
        d0 = q * CH
        for c in range(8):
            for v in range(4):
                acc[pl.ds(a0 + c * CH + 16 * v, 16)] = zf
        for v in range(4):
            den[pl.ds(d0 + 16 * v, 16)] = zf

        def pb_(p, _):
            @pl.when((word >> p) & 1 != 0)
            def _():
                g = (q * P + p) * GSLOT
                pltpu.make_async_copy(table.at[pl.ds(0, 2 * CH)],
                                      G.at[pl.ds(g, 2 * CH)], sem_g).wait()
                pltpu.make_async_copy(table.at[pl.ds(0, 2 * CH)],
                                      G.at[pl.ds(g + 2 * CH, 2 * CH)],
                                      sem_g).wait()
                s = (qa * P + p) * CH
                for v in range(4):
                    pix = lane + (16 * v)
                    tw = []
                    for k in range(4):
                        wk = wb[pl.ds(s * 4 + k * CH + 16 * v, 16)]
                        tw.append(jnp.where(wk > THR, wk, 0.0))
                    dn = pl.ds(d0 + 16 * v, 16)
                    den[dn] = den[dn] + tw[0] + tw[1] + tw[2] + tw[3]
                    for c in range(8):
                        cv = jnp.full((16,), c, jnp.int32)
                        o = pl.ds(a0 + c * CH + 16 * v, 16)
                        sacc = acc[o]
                        for k in range(4):
                            val = plsc.load_gather(G, [g + k * CH + pix, cv])
                            sacc = sacc + tw[k] * val
                        acc[o] = sacc

        lax.fori_loop(0, P, pb_, None)

        for v in range(4):
            dv = jnp.maximum(den[pl.ds(d0 + 16 * v, 16)], EPS)
            for c in range(8):
                o = pl.ds(a0 + c * CH + 16 * v, 16)
                acc[o] = acc[o] / dv
        for c in range(8):
            pltpu.async_copy(acc.at[pl.ds(a0 + c * CH, CH)],
                             out.at[pl.ds(c * NPIX + obase, CH)], sem_out)

    def drain_out():
        for _ in range(8):
            pltpu.make_async_copy(acc.at[pl.ds(0, CH)],
                                  out.at[pl.ds(0, CH)], sem_out).wait()

    fire_aux(0)
    fire_gathers(0, 0)

    def main_body(i, _):
        @pl.when(i < CPW)
        def _():
            @pl.when(i > 0)
            def _():
                fire_gathers(i, lax.rem(i, 2))

            @pl.when(i + 1 < CPW)
            def _():
                fire_aux(i + 1)

        @pl.when(i > 0)
        def _():
            @pl.when(i > 2)
            def _():
                drain_out()

            compute(i - 1, lax.rem(i - 1, 2))

    lax.fori_loop(0, CPW + 1, main_body, None)
    drain_out()
    drain_out()


_pers2equi_sc = functools.partial(
    pl.kernel,
    out_type=jax.ShapeDtypeStruct((8 * NPIX,), jnp.float32),
    mesh=plsc.VectorSubcoreMesh(core_axis_name="c", subcore_axis_name="s"),
    compiler_params=pltpu.CompilerParams(needs_layout_passes=False,
                                         use_tc_tiling_on_sc=False),
    scratch_types=[
        pltpu.VMEM((2 * P * GSLOT, 8), jnp.float32),
        pltpu.VMEM((2 * P * 2, 2 * CH), jnp.int32),
        pltpu.VMEM((3 * P * CH,), jnp.int32),
        pltpu.VMEM((3 * P * CH,), jnp.int32),
        pltpu.VMEM((3 * P * CH * 4,), jnp.float32),
        pltpu.VMEM((NCHUNK + 16,), jnp.int32),
        pltpu.VMEM((2 * 8 * CH,), jnp.float32),
        pltpu.VMEM((2 * CH,), jnp.float32),
        pltpu.SemaphoreType.DMA,
        pltpu.SemaphoreType.DMA,
        pltpu.SemaphoreType.DMA,
    ],
)(_sc_kernel)


def kernel(x, w_list, mask, x0, y0, x1, y1):
    del x1, y1
    xv = jnp.transpose(x, (0, 1, 4, 2, 3))[0]
    table = _transpose_tc(xv)
    wf = jnp.transpose(w_list.reshape(P, H, 4, 128, 4),
                       (0, 1, 2, 4, 3)).reshape(P * NPIX * 4)
    x0f = x0.astype(jnp.int32).reshape(P * NPIX)
    y0f = y0.astype(jnp.int32).reshape(P * NPIX)
    bits = mask.astype(jnp.int32).reshape(P, NCHUNK, CH).max(-1)
    bmpack = (bits << jnp.arange(P, dtype=jnp.int32)[:, None]).sum(0)
    outf = _pers2equi_sc(table, wf, x0f, y0f, bmpack)
    out = outf.reshape(8, H // 8, 4, 8, 128).transpose(0, 1, 3, 2, 4)
    return out.reshape(1, 8, H, W)


# speedup vs baseline: 4.5302x; 1.0595x over previous
import functools

import jax
import jax.numpy as jnp
from jax import lax
from jax.experimental import pallas as pl
from jax.experimental.pallas import tpu as pltpu
from jax.experimental.pallas import tpu_sc as plsc

P = 18
PH = 224
H, W = 256, 512
NPIX = H * W
NT = PH * PH * P
CH = 64
NCHUNK = NPIX // CH
NC, NS = 2, 16
NW = NC * NS
CPW = NCHUNK // NW
GSLOT = 4 * CH
THR = 1e-5
EPS = 1e-12

TYB = 8


def _tp_body(x_ref, o_ref):
    for y in range(TYB):
        o_ref[pl.ds(y * PH, PH), :] = x_ref[:, 0, y, :].T


_transpose_tc = pl.pallas_call(
    _tp_body,
    grid=(P, PH // TYB),
    in_specs=[pl.BlockSpec((8, 1, TYB, PH), lambda p, i: (0, p, i, 0))],
    out_specs=pl.BlockSpec((TYB * PH, 8), lambda p, i: (p * (PH // TYB) + i, 0)),
    out_shape=jax.ShapeDtypeStruct((NT, 8), jnp.float32),
)


def _sc_kernel(table, wf, x0f, y0f, bmpack, out,
               G, idxb, x0b, y0b, wb, bmw, acc, den,
               sem_aux, sem_g, sem_out):
    wid = lax.axis_index("s") * NC + lax.axis_index("c")
    lane = lax.iota(jnp.int32, 16)
    zf = jnp.zeros((16,), jnp.float32)

    pltpu.sync_copy(bmpack.at[pl.ds(0, NCHUNK)], bmw.at[pl.ds(0, NCHUNK)])

    def word_of(ci):
        return bmw[pl.ds(wid + ci * NW, 16)][0]

    def fire_aux(ci):
        chunk = wid + ci * NW
        base = chunk * CH
        row = chunk >> 3
        s8 = chunk & 7
        jhi = s8 >> 1
        jlo = (s8 & 1) * CH
        qa = lax.rem(ci, 3)
        word = word_of(ci)

        def pb_(p, _):
            @pl.when((word >> p) & 1 != 0)
            def _():
                s = (qa * P + p) * CH
                pltpu.async_copy(x0f.at[pl.ds(p * NPIX + base, CH)],
                                 x0b.at[pl.ds(s, CH)], sem_aux)
                pltpu.async_copy(y0f.at[pl.ds(p * NPIX + base, CH)],
                                 y0b.at[pl.ds(s, CH)], sem_aux)
                woff = ((p * H + row) * 4 + jhi) * 512 + jlo
                for k in range(4):
                    pltpu.async_copy(wf.at[pl.ds(woff + k * 128, CH)],
                                     wb.at[pl.ds(s * 4 + k * CH, CH)], sem_aux)

        lax.fori_loop(0, P, pb_, None)

    def fire_gathers(ci, q):
        qa = lax.rem(ci, 3)
        word = word_of(ci)

        def pb_(p, _):
            @pl.when((word >> p) & 1 != 0)
            def _():
                s = (qa * P + p) * CH
                pltpu.make_async_copy(x0f.at[pl.ds(0, CH)],
                                      x0b.at[pl.ds(s, CH)], sem_aux).wait()
                pltpu.make_async_copy(y0f.at[pl.ds(0, CH)],
                                      y0b.at[pl.ds(s, CH)], sem_aux).wait()
                for k in range(4):
                    pltpu.make_async_copy(wf.at[pl.ds(0, CH)],
                                          wb.at[pl.ds(s * 4 + k * CH, CH)],
                                          sem_aux).wait()
                r = (q * P + p) * 2
                pb = p * (PH * PH)
                for v in range(4):
                    xx0 = x0b[pl.ds(s + 16 * v, 16)]
                    yy0 = y0b[pl.ds(s + 16 * v, 16)]
                    xc0 = pb + xx0
                    xc1 = pb + jnp.minimum(xx0 + 1, PH - 1)
                    yr0 = yy0 * PH
                    yr1 = jnp.minimum(yy0 + 1, PH - 1) * PH
                    idxb[r, pl.ds(16 * v, 16)] = yr0 + xc0
                    idxb[r, pl.ds(CH + 16 * v, 16)] = yr1 + xc0
                    idxb[r + 1, pl.ds(16 * v, 16)] = yr0 + xc1
                    idxb[r + 1, pl.ds(CH + 16 * v, 16)] = yr1 + xc1
                g = (q * P + p) * GSLOT
                pltpu.async_copy(table.at[idxb.at[r]],
                                 G.at[pl.ds(g, 2 * CH)], sem_g)
                pltpu.async_copy(table.at[idxb.at[r + 1]],
                                 G.at[pl.ds(g + 2 * CH, 2 * CH)], sem_g)

        lax.fori_loop(0, P, pb_, None)

    def compute(ci, q):
        chunk = wid + ci * NW
        row = chunk >> 3
        s8 = chunk & 7
        obase = ((row >> 3) * 4 + (s8 >> 1)) * 1024 + (row & 7) * 128 + (s8 & 1) * CH
        qa = lax.rem(ci, 3)
        word = word_of(ci)
        a0 = q * (8 * CH)
        d0 = q * CH
        for c in range(8):
            for v in range(4):
                acc[pl.ds(a0 + c * CH + 16 * v, 16)] = zf
        for v in range(4):
            den[pl.ds(d0 + 16 * v, 16)] = zf

        def pb_(p, _):
            @pl.when((word >> p) & 1 != 0)
            def _():
                g = (q * P + p) * GSLOT
                pltpu.make_async_copy(table.at[pl.ds(0, 2 * CH)],
                                      G.at[pl.ds(g, 2 * CH)], sem_g).wait()
                pltpu.make_async_copy(table.at[pl.ds(0, 2 * CH)],
                                      G.at[pl.ds(g + 2 * CH, 2 * CH)],
                                      sem_g).wait()
                s = (qa * P + p) * CH
                for v in range(4):
                    pix = lane + (16 * v)
                    tw = []
                    for k in range(4):
                        wk = wb[pl.ds(s * 4 + k * CH + 16 * v, 16)]
                        tw.append(jnp.where(wk > THR, wk, 0.0))
                    dn = pl.ds(d0 + 16 * v, 16)
                    den[dn] = den[dn] + tw[0] + tw[1] + tw[2] + tw[3]
                    for c in range(8):
                        cv = jnp.full((16,), c, jnp.int32)
                        o = pl.ds(a0 + c * CH + 16 * v, 16)
                        sacc = acc[o]
                        for k in range(4):
                            val = plsc.load_gather(G, [g + k * CH + pix, cv])
                            sacc = sacc + tw[k] * val
                        acc[o] = sacc

        lax.fori_loop(0, P, pb_, None)

        for v in range(4):
            dv = jnp.maximum(den[pl.ds(d0 + 16 * v, 16)], EPS)
            for c in range(8):
                o = pl.ds(a0 + c * CH + 16 * v, 16)
                acc[o] = acc[o] / dv
        for c in range(8):
            pltpu.async_copy(acc.at[pl.ds(a0 + c * CH, CH)],
                             out.at[pl.ds(c * NPIX + obase, CH)], sem_out)

    def drain_out():
        for _ in range(8):
            pltpu.make_async_copy(acc.at[pl.ds(0, CH)],
                                  out.at[pl.ds(0, CH)], sem_out).wait()

    fire_aux(0)
    fire_gathers(0, 0)

    def main_body(i, _):
        @pl.when(i < CPW)
        def _():
            @pl.when(i > 0)
            def _():
                fire_gathers(i, lax.rem(i, 2))

            @pl.when(i + 1 < CPW)
            def _():
                fire_aux(i + 1)

        @pl.when(i > 0)
        def _():
            @pl.when(i > 2)
            def _():
                drain_out()

            compute(i - 1, lax.rem(i - 1, 2))

    lax.fori_loop(0, CPW + 1, main_body, None)
    drain_out()
    drain_out()


_pers2equi_sc = functools.partial(
    pl.kernel,
    out_type=jax.ShapeDtypeStruct((8 * NPIX,), jnp.float32),
    mesh=plsc.VectorSubcoreMesh(core_axis_name="c", subcore_axis_name="s"),
    compiler_params=pltpu.CompilerParams(needs_layout_passes=False,
                                         use_tc_tiling_on_sc=False),
    scratch_types=[
        pltpu.VMEM((2 * P * GSLOT, 8), jnp.float32),
        pltpu.VMEM((2 * P * 2, 2 * CH), jnp.int32),
        pltpu.VMEM((3 * P * CH,), jnp.int32),
        pltpu.VMEM((3 * P * CH,), jnp.int32),
        pltpu.VMEM((3 * P * CH * 4,), jnp.float32),
        pltpu.VMEM((NCHUNK + 16,), jnp.int32),
        pltpu.VMEM((2 * 8 * CH,), jnp.float32),
        pltpu.VMEM((2 * CH,), jnp.float32),
        pltpu.SemaphoreType.DMA,
        pltpu.SemaphoreType.DMA,
        pltpu.SemaphoreType.DMA,
    ],
)(_sc_kernel)


def kernel(x, w_list, mask, x0, y0, x1, y1):
    del x1, y1
    xv = jnp.transpose(x, (0, 1, 4, 2, 3))[0]
    table = _transpose_tc(xv)
    wf = jnp.transpose(w_list.reshape(P, H, 4, 128, 4),
                       (0, 1, 2, 4, 3)).reshape(P * NPIX * 4)
    x0f = x0.astype(jnp.int32).reshape(P * NPIX)
    y0f = y0.astype(jnp.int32).reshape(P * NPIX)
    bits = mask.astype(jnp.int32).reshape(P, NCHUNK, CH).max(-1)
    bmpack = (bits << jnp.arange(P, dtype=jnp.int32)[:, None]).sum(0)
    outf = _pers2equi_sc(table, wf, x0f, y0f, bmpack)
    out = outf.reshape(8, H // 8, 4, 8, 128).transpose(0, 1, 3, 2, 4)
    return out.reshape(1, 8, H, W)
